# baseline (speedup 1.0000x reference)
import functools

import jax
import jax.numpy as jnp
from jax import lax
from jax.experimental import pallas as pl
from jax.experimental.pallas import tpu as pltpu
from jax.experimental.pallas import tpu_sc as plsc


def _sc_gather(table, idx):
    (n,) = idx.shape
    info = plsc.get_sparse_core_info()
    nw = info.num_subcores
    b_per_w = n // nw
    mesh = plsc.VectorSubcoreMesh(
        core_axis_name="c", subcore_axis_name="s", num_cores=1)

    @functools.partial(
        pl.kernel,
        mesh=mesh,
        out_type=jax.ShapeDtypeStruct((n,), jnp.float32),
        scratch_types=[
            pltpu.VMEM((b_per_w,), jnp.int32),
            pltpu.VMEM((b_per_w,), jnp.float32),
            pltpu.SemaphoreType.DMA,
        ],
    )
    def k(table_hbm, idx_hbm, out_hbm, idx_v, rows_v, sem):
        wid = lax.axis_index("s")
        base = wid * b_per_w
        pltpu.sync_copy(idx_hbm.at[pl.ds(base, b_per_w)], idx_v)
        pltpu.async_copy(table_hbm.at[idx_v], rows_v, sem).wait()
        pltpu.sync_copy(rows_v, out_hbm.at[pl.ds(base, b_per_w)])

    return k(table, idx)


def _dense_body(u_ref, p_ref, t_ref, w1_ref, w2_ref, o_ref):
    t = t_ref[...]
    d1 = u_ref[...] - t
    d2 = p_ref[...] - t
    sq = d1 * d1 * w1_ref[...] + d2 * d2 * w2_ref[...]
    o_ref[...] = jnp.sum(sq, axis=0)


def _add_body(a_ref, b_ref, o_ref):
    o_ref[...] = a_ref[...] + b_ref[...]


def kernel(user_ebs, playlist_ebs, track_ebs, track_entity_ids, B1, B2, track_biases):
    batch, eb = user_ebs.shape
    bias = _sc_gather(track_biases, track_entity_ids.astype(jnp.int32))

    grid = 16
    blk = batch // grid
    w1 = (B1 * B1).reshape(eb, 1)
    w2 = (B2 * B2).reshape(eb, 1)
    o12 = pl.pallas_call(
        _dense_body,
        grid=(grid,),
        in_specs=[
            pl.BlockSpec((eb, blk), lambda i: (0, i)),
            pl.BlockSpec((eb, blk), lambda i: (0, i)),
            pl.BlockSpec((eb, blk), lambda i: (0, i)),
            pl.BlockSpec((eb, 1), lambda i: (0, 0)),
            pl.BlockSpec((eb, 1), lambda i: (0, 0)),
        ],
        out_specs=pl.BlockSpec((blk,), lambda i: (i,)),
        out_shape=jax.ShapeDtypeStruct((batch,), jnp.float32),
    )(user_ebs.T, playlist_ebs.T, track_ebs.T, w1, w2)

    out = pl.pallas_call(
        _add_body,
        in_specs=[
            pl.BlockSpec((batch,), lambda: (0,)),
            pl.BlockSpec((batch,), lambda: (0,)),
        ],
        out_specs=pl.BlockSpec((batch,), lambda: (0,)),
        out_shape=jax.ShapeDtypeStruct((batch,), jnp.float32),
    )(o12, bias)
    return out


# speedup vs baseline: 1.1902x; 1.1902x over previous
import functools

import jax
import jax.numpy as jnp
from jax import lax
from jax.experimental import pallas as pl
from jax.experimental.pallas import tpu as pltpu
from jax.experimental.pallas import tpu_sc as plsc

_SLABS = 8


def _sc_gather(table, idx):
    (n,) = idx.shape
    info = plsc.get_sparse_core_info()
    nw = info.num_subcores
    b_per_w = n // nw
    mesh = plsc.VectorSubcoreMesh(
        core_axis_name="c", subcore_axis_name="s", num_cores=1)

    @functools.partial(
        pl.kernel,
        mesh=mesh,
        out_type=jax.ShapeDtypeStruct((n,), jnp.float32),
        scratch_types=[
            pltpu.VMEM((b_per_w,), jnp.int32),
            pltpu.VMEM((b_per_w,), jnp.float32),
            pltpu.SemaphoreType.DMA,
        ],
    )
    def k(table_hbm, idx_hbm, out_hbm, idx_v, rows_v, sem):
        wid = lax.axis_index("s")
        base = wid * b_per_w
        pltpu.sync_copy(idx_hbm.at[pl.ds(base, b_per_w)], idx_v)
        pltpu.async_copy(table_hbm.at[idx_v], rows_v, sem).wait()
        pltpu.sync_copy(rows_v, out_hbm.at[pl.ds(base, b_per_w)])

    return k(table, idx)


def _dense_body(u_hbm, p_hbm, t_hbm, w1_ref, w2_ref, o_ref, ub, pb, tb, sems):
    eb = ub.shape[0]
    rows = eb // _SLABS

    def slab_copies(i):
        sl = pl.ds(i * rows, rows)
        return (
            pltpu.make_async_copy(u_hbm.at[sl], ub.at[sl], sems.at[0, i]),
            pltpu.make_async_copy(p_hbm.at[sl], pb.at[sl], sems.at[1, i]),
            pltpu.make_async_copy(t_hbm.at[sl], tb.at[sl], sems.at[2, i]),
        )

    for i in range(_SLABS):
        for c in slab_copies(i):
            c.start()
    for i in range(_SLABS):
        for c in slab_copies(i):
            c.wait()
        sl = pl.ds(i * rows, rows)
        t = tb[sl]
        d1 = ub[sl] - t
        d2 = pb[sl] - t
        sq = d1 * d1 * w1_ref[sl] + d2 * d2 * w2_ref[sl]
        partial = jnp.sum(sq, axis=0)
        if i == 0:
            o_ref[...] = partial
        else:
            o_ref[...] += partial


def _add_body(a_ref, b_ref, o_ref):
    o_ref[...] = a_ref[...] + b_ref[...]


def kernel(user_ebs, playlist_ebs, track_ebs, track_entity_ids, B1, B2, track_biases):
    batch, eb = user_ebs.shape
    bias = _sc_gather(track_biases, track_entity_ids.astype(jnp.int32))

    w1 = (B1 * B1).reshape(eb, 1)
    w2 = (B2 * B2).reshape(eb, 1)
    o12 = pl.pallas_call(
        _dense_body,
        in_specs=[
            pl.BlockSpec(memory_space=pltpu.MemorySpace.HBM),
            pl.BlockSpec(memory_space=pltpu.MemorySpace.HBM),
            pl.BlockSpec(memory_space=pltpu.MemorySpace.HBM),
            pl.BlockSpec((eb, 1), lambda: (0, 0)),
            pl.BlockSpec((eb, 1), lambda: (0, 0)),
        ],
        out_specs=pl.BlockSpec((batch,), lambda: (0,)),
        out_shape=jax.ShapeDtypeStruct((batch,), jnp.float32),
        scratch_shapes=[
            pltpu.VMEM((eb, batch), jnp.float32),
            pltpu.VMEM((eb, batch), jnp.float32),
            pltpu.VMEM((eb, batch), jnp.float32),
            pltpu.SemaphoreType.DMA((3, _SLABS)),
        ],
    )(user_ebs.T, playlist_ebs.T, track_ebs.T, w1, w2)

    out = pl.pallas_call(
        _add_body,
        in_specs=[
            pl.BlockSpec((batch,), lambda: (0,)),
            pl.BlockSpec((batch,), lambda: (0,)),
        ],
        out_specs=pl.BlockSpec((batch,), lambda: (0,)),
        out_shape=jax.ShapeDtypeStruct((batch,), jnp.float32),
    )(o12, bias)
    return out


# speedup vs baseline: 1.2764x; 1.0724x over previous
import functools

import jax
import jax.numpy as jnp
from jax import lax
from jax.experimental import pallas as pl
from jax.experimental.pallas import tpu as pltpu
from jax.experimental.pallas import tpu_sc as plsc

_SLABS = 8


def _sc_gather(table, idx):
    (n,) = idx.shape
    info = plsc.get_sparse_core_info()
    nw = info.num_subcores
    b_per_w = n // nw
    mesh = plsc.VectorSubcoreMesh(
        core_axis_name="c", subcore_axis_name="s", num_cores=1)

    @functools.partial(
        pl.kernel,
        mesh=mesh,
        out_type=jax.ShapeDtypeStruct((n,), jnp.float32),
        scratch_types=[
            pltpu.VMEM((b_per_w,), jnp.int32),
            pltpu.VMEM((b_per_w,), jnp.float32),
            pltpu.SemaphoreType.DMA,
        ],
    )
    def k(table_hbm, idx_hbm, out_hbm, idx_v, rows_v, sem):
        wid = lax.axis_index("s")
        base = wid * b_per_w
        pltpu.sync_copy(idx_hbm.at[pl.ds(base, b_per_w)], idx_v)
        pltpu.async_copy(table_hbm.at[idx_v], rows_v, sem).wait()
        pltpu.sync_copy(rows_v, out_hbm.at[pl.ds(base, b_per_w)])

    return k(table, idx)


def _dense_body(u_hbm, p_hbm, t_hbm, b1_ref, b2_ref, o_ref, ub, pb, tb, sems):
    eb = ub.shape[0]
    rows = eb // _SLABS
    b1 = b1_ref[...]
    b2 = b2_ref[...]
    w1 = b1 * b1
    w2 = b2 * b2

    def slab_copies(i):
        sl = pl.ds(i * rows, rows)
        return (
            pltpu.make_async_copy(u_hbm.at[sl], ub.at[sl], sems.at[0, i]),
            pltpu.make_async_copy(p_hbm.at[sl], pb.at[sl], sems.at[1, i]),
            pltpu.make_async_copy(t_hbm.at[sl], tb.at[sl], sems.at[2, i]),
        )

    for i in range(_SLABS):
        for c in slab_copies(i):
            c.start()
    for i in range(_SLABS):
        for c in slab_copies(i):
            c.wait()
        sl = pl.ds(i * rows, rows)
        t = tb[sl]
        d1 = ub[sl] - t
        d2 = pb[sl] - t
        w1c = w1[i * rows:(i + 1) * rows].reshape(rows, 1)
        w2c = w2[i * rows:(i + 1) * rows].reshape(rows, 1)
        sq = d1 * d1 * w1c + d2 * d2 * w2c
        partial = jnp.sum(sq, axis=0)
        if i == 0:
            o_ref[...] = partial
        else:
            o_ref[...] += partial


def _add_body(a_ref, b_ref, o_ref):
    o_ref[...] = a_ref[...] + b_ref[...]


def kernel(user_ebs, playlist_ebs, track_ebs, track_entity_ids, B1, B2, track_biases):
    batch, eb = user_ebs.shape
    bias = _sc_gather(track_biases, track_entity_ids.astype(jnp.int32))

    o12 = pl.pallas_call(
        _dense_body,
        in_specs=[
            pl.BlockSpec(memory_space=pltpu.MemorySpace.HBM),
            pl.BlockSpec(memory_space=pltpu.MemorySpace.HBM),
            pl.BlockSpec(memory_space=pltpu.MemorySpace.HBM),
            pl.BlockSpec((eb,), lambda: (0,)),
            pl.BlockSpec((eb,), lambda: (0,)),
        ],
        out_specs=pl.BlockSpec((batch,), lambda: (0,)),
        out_shape=jax.ShapeDtypeStruct((batch,), jnp.float32),
        scratch_shapes=[
            pltpu.VMEM((eb, batch), jnp.float32),
            pltpu.VMEM((eb, batch), jnp.float32),
            pltpu.VMEM((eb, batch), jnp.float32),
            pltpu.SemaphoreType.DMA((3, _SLABS)),
        ],
    )(user_ebs.T, playlist_ebs.T, track_ebs.T, B1, B2)

    out = pl.pallas_call(
        _add_body,
        in_specs=[
            pl.BlockSpec((batch,), lambda: (0,)),
            pl.BlockSpec((batch,), lambda: (0,)),
        ],
        out_specs=pl.BlockSpec((batch,), lambda: (0,)),
        out_shape=jax.ShapeDtypeStruct((batch,), jnp.float32),
    )(o12, bias)
    return out
